# baseline (speedup 1.0000x reference)
import functools

import jax
import jax.numpy as jnp
from jax import lax
from jax.experimental import pallas as pl

INF = 100000.0
DIM = 96
RES = 32
NUM_PATCH = RES * RES
K = 10
BATCH = 32
RB = 256
NB = NUM_PATCH // RB
NCH = NUM_PATCH // 128

_SORT8 = [
    (0, 1), (2, 3), (4, 5), (6, 7),
    (0, 2), (1, 3), (4, 6), (5, 7),
    (1, 2), (5, 6),
    (0, 4), (1, 5), (2, 6), (3, 7),
    (2, 4), (3, 5),
    (1, 2), (3, 4), (5, 6),
]


def _body(feat_ref, rel_ref, out_ref):
    r = pl.program_id(0)
    b = pl.program_id(1)
    x = feat_ref[0]
    nrm = jnp.sqrt(jnp.sum(x * x, axis=1, keepdims=True))
    xn = x / jnp.clip(nrm, 1e-12, None)
    s = jnp.sum(xn * xn, axis=1)
    xr_raw = feat_ref[0, pl.ds(r * RB, RB), :]
    nrm_r = jnp.sqrt(jnp.sum(xr_raw * xr_raw, axis=1, keepdims=True))
    xr = xr_raw / jnp.clip(nrm_r, 1e-12, None)
    sr = jnp.sum(xr * xr, axis=1)
    prod = lax.dot_general(xr, xn, (((1,), (1,)), ((), ())),
                           preferred_element_type=jnp.float32)
    dist = sr[:, None] + s[None, :] - 2.0 * prod + rel_ref[0]
    gi = r * RB + lax.broadcasted_iota(jnp.int32, (RB, NUM_PATCH), 0)
    gj = lax.broadcasted_iota(jnp.int32, (RB, NUM_PATCH), 1)
    nbr = (jnp.abs((gi >> 5) - (gj >> 5)) <= 1) & (jnp.abs((gi & 31) - (gj & 31)) <= 1)
    dist = jnp.where(nbr, dist + INF, dist)

    bits = lax.bitcast_convert_type(dist, jnp.int32)
    key = jnp.where(bits < 0, bits ^ jnp.int32(0x7FFFFFFF), bits)
    packed = (key & jnp.int32(-8)) | (gj >> 7)

    ch = [packed[:, c * 128:(c + 1) * 128] for c in range(NCH)]
    for a, c in _SORT8:
        lo = jnp.minimum(ch[a], ch[c])
        hi = jnp.maximum(ch[a], ch[c])
        ch[a], ch[c] = lo, hi
    sentinel = jnp.full((RB, 128), jnp.int32(0x7FFFFFFF))
    ch.append(sentinel)

    lane = lax.broadcasted_iota(jnp.int32, (RB, 128), 1)
    outs = []
    for _ in range(K):
        m = jnp.min(ch[0], axis=1)
        l_star = jnp.min(jnp.where(ch[0] == m[:, None], lane, jnp.int32(1 << 30)),
                         axis=1)
        j = ((m & 7) << 7) | l_star
        outs.append(j + b * NUM_PATCH)
        sel = lane == l_star[:, None]
        for lv in range(NCH):
            ch[lv] = jnp.where(sel, ch[lv + 1], ch[lv])
    out_ref[0] = jnp.stack(outs, axis=1)


@functools.partial(jax.jit, static_argnums=())
def _topk_call(node_feature, relative_pos):
    return pl.pallas_call(
        _body,
        grid=(NB, BATCH),
        in_specs=[
            pl.BlockSpec((1, NUM_PATCH, DIM), lambda r, b: (b, 0, 0)),
            pl.BlockSpec((1, RB, NUM_PATCH), lambda r, b: (0, r, 0)),
        ],
        out_specs=pl.BlockSpec((1, RB, K), lambda r, b: (b, r, 0)),
        out_shape=jax.ShapeDtypeStruct((BATCH, NUM_PATCH, K), jnp.int32),
    )(node_feature, relative_pos)


def kernel(node_feature, relative_pos):
    b, n, _ = node_feature.shape
    tk = _topk_call(node_feature, relative_pos)
    src = jnp.broadcast_to(
        jnp.arange(b * n, dtype=jnp.int32).reshape(b, n, 1), (b, n, K))
    edge_list = jnp.stack([tk, src], axis=-1).reshape(-1, 2)
    relation = jnp.zeros((edge_list.shape[0], 1), dtype=edge_list.dtype)
    edge_list = jnp.concatenate([edge_list, relation], axis=-1)
    return (edge_list, 1)


# speedup vs baseline: 2.2268x; 2.2268x over previous
import functools

import jax
import jax.numpy as jnp
from jax import lax
from jax.experimental import pallas as pl
from jax.experimental.pallas import tpu as pltpu

INF = 100000.0
DIM = 96
RES = 32
NUM_PATCH = RES * RES
K = 10
BATCH = 32
RB = 256
NB = NUM_PATCH // RB
NCH = NUM_PATCH // 128

_SORT8 = [
    (0, 1), (2, 3), (4, 5), (6, 7),
    (0, 2), (1, 3), (4, 6), (5, 7),
    (1, 2), (5, 6),
    (0, 4), (1, 5), (2, 6), (3, 7),
    (2, 4), (3, 5),
    (1, 2), (3, 4), (5, 6),
]


def _norm_body(feat_ref, out_ref):
    x = feat_ref[0]
    nrm = jnp.sqrt(jnp.sum(x * x, axis=1, keepdims=True))
    out_ref[0] = x / jnp.clip(nrm, 1e-12, None)


def _body(xn_ref, rel_ref, out_ref, bias_ref):
    r = pl.program_id(0)
    b = pl.program_id(1)

    @pl.when(b == 0)
    def _():
        j0 = lax.broadcasted_iota(jnp.int32, (NUM_PATCH, RB), 0)
        i0 = r * RB + lax.broadcasted_iota(jnp.int32, (NUM_PATCH, RB), 1)
        nbr = ((jnp.abs((j0 >> 5) - (i0 >> 5)) <= 1)
               & (jnp.abs((j0 & 31) - (i0 & 31)) <= 1))
        bias_ref[...] = rel_ref[0] + jnp.where(nbr, INF, 0.0) + 2.0

    xn = xn_ref[0]
    xr = xn_ref[0, pl.ds(r * RB, RB), :]
    prod = lax.dot_general(xn, xr, (((1,), (1,)), ((), ())),
                           preferred_element_type=jnp.float32)
    d = bias_ref[...] - 2.0 * prod
    ib = lax.bitcast_convert_type(d, jnp.int32)
    ch = []
    for c in range(NCH):
        sl = (ib[c * 128:(c + 1) * 128, :] & jnp.int32(-8)) | jnp.int32(c)
        ch.append(lax.bitcast_convert_type(sl, jnp.float32))
    ch.append(jnp.full((128, RB), jnp.inf, dtype=jnp.float32))

    for a, c in _SORT8:
        lo = jnp.minimum(ch[a], ch[c])
        hi = jnp.maximum(ch[a], ch[c])
        ch[a], ch[c] = lo, hi

    iota0 = lax.broadcasted_iota(jnp.int32, (128, RB), 0)
    outs = []
    for k in range(K):
        m = jnp.min(ch[0], axis=0)
        jm = jnp.min(jnp.where(ch[0] == m[None, :], iota0, jnp.int32(1 << 30)),
                     axis=0)
        mc = lax.bitcast_convert_type(m, jnp.int32) & 7
        outs.append(mc * 128 + jm + b * NUM_PATCH)
        upper = min(NCH, K - 1 - k)
        if upper > 0:
            sel = iota0 == jm[None, :]
            for lv in range(upper):
                ch[lv] = jnp.where(sel, ch[lv + 1], ch[lv])
    out_ref[0] = jnp.stack(outs, axis=1)


@functools.partial(jax.jit, static_argnums=())
def _topk_call(node_feature, relative_pos):
    xn = pl.pallas_call(
        _norm_body,
        grid=(BATCH,),
        in_specs=[pl.BlockSpec((1, NUM_PATCH, DIM), lambda b: (b, 0, 0))],
        out_specs=pl.BlockSpec((1, NUM_PATCH, DIM), lambda b: (b, 0, 0)),
        out_shape=jax.ShapeDtypeStruct((BATCH, NUM_PATCH, DIM), jnp.float32),
    )(node_feature)
    return pl.pallas_call(
        _body,
        grid=(NB, BATCH),
        in_specs=[
            pl.BlockSpec((1, NUM_PATCH, DIM), lambda r, b: (b, 0, 0)),
            pl.BlockSpec((1, NUM_PATCH, RB), lambda r, b: (0, 0, r)),
        ],
        out_specs=pl.BlockSpec((1, RB, K), lambda r, b: (b, r, 0)),
        out_shape=jax.ShapeDtypeStruct((BATCH, NUM_PATCH, K), jnp.int32),
        scratch_shapes=[pltpu.VMEM((NUM_PATCH, RB), jnp.float32)],
    )(xn, relative_pos)


def kernel(node_feature, relative_pos):
    b, n, _ = node_feature.shape
    tk = _topk_call(node_feature, relative_pos)
    src = jnp.broadcast_to(
        jnp.arange(b * n, dtype=jnp.int32).reshape(b, n, 1), (b, n, K))
    edge_list = jnp.stack([tk, src], axis=-1).reshape(-1, 2)
    relation = jnp.zeros((edge_list.shape[0], 1), dtype=edge_list.dtype)
    edge_list = jnp.concatenate([edge_list, relation], axis=-1)
    return (edge_list, 1)
